# baseline (speedup 1.0000x reference)
import functools

import jax
import jax.numpy as jnp
from jax import lax
from jax.experimental import pallas as pl
from jax.experimental.pallas import tpu as pltpu
from jax.experimental.pallas import tpu_sc as plsc

D = 1024
L = 16
NW = 32
ROWS = 32768
ROWS_PER_W = ROWS // NW
K = 32
NCHUNK = ROWS_PER_W // K
S_LEN = 8192

_mesh = plsc.VectorSubcoreMesh(core_axis_name="c", subcore_axis_name="s")


@functools.partial(
    pl.kernel,
    mesh=_mesh,
    out_type=jax.ShapeDtypeStruct((ROWS, D), jnp.float32),
    scratch_types=[
        pltpu.VMEM((NCHUNK, K), jnp.int32),
        pltpu.VMEM((K, D), jnp.float32),
        pltpu.VMEM((K, D), jnp.float32),
        pltpu.SemaphoreType.DMA,
    ],
)
def _emb_kernel(idx_hbm, tok_hbm, pos_hbm, out_hbm, idx_v, rows_v, pos_v, sem):
    cid = lax.axis_index("c")
    sid = lax.axis_index("s")
    wid = sid * 2 + cid
    base = wid * ROWS_PER_W
    s0 = lax.rem(base, S_LEN)

    pltpu.sync_copy(idx_hbm.at[pl.ds(wid * NCHUNK, NCHUNK)], idx_v)

    def chunk_body(ci, carry):
        pltpu.async_copy(tok_hbm.at[idx_v.at[ci]], rows_v, sem).wait()
        pltpu.sync_copy(pos_hbm.at[pl.ds(s0 + ci * K, K)], pos_v)

        def row_body(i, c2):
            for j in range(D // L):
                sl = pl.ds(j * L, L)
                rows_v[i, sl] = rows_v[i, sl] + pos_v[i, sl]
            return c2

        lax.fori_loop(0, K, row_body, 0)
        pltpu.sync_copy(rows_v, out_hbm.at[pl.ds(base + ci * K, K)])
        return carry

    lax.fori_loop(0, NCHUNK, chunk_body, 0)


def kernel(x, token_table, pos_table):
    b, s = x.shape
    idx = x.reshape(ROWS).astype(jnp.int32).reshape(NW * NCHUNK, K)
    out = _emb_kernel(idx, token_table, pos_table)
    return out.reshape(b, s, D)
